# initial kernel scaffold (unmeasured)
import jax
import jax.numpy as jnp
from jax import lax
from jax.experimental import pallas as pl
from jax.experimental.pallas import tpu as pltpu


def kernel(
    x,
):
    def body(*refs):
        pass

    out_shape = jax.ShapeDtypeStruct(..., jnp.float32)
    return pl.pallas_call(body, out_shape=out_shape)(...)


# baseline (device time: 23562 ns/iter reference)
import jax
import jax.numpy as jnp
from jax import lax
from jax.experimental import pallas as pl
from jax.experimental.pallas import tpu as pltpu

N_DEV = 4
N_HOP = N_DEV - 1


def kernel(x):
    m, n = x.shape
    mc = m // N_DEV

    def body(x_ref, out_ref, rs_send, rs_recv, ag_buf, send_sems, recv_sems):
        my = lax.axis_index("i")
        left = lax.rem(my + N_DEV - 1, N_DEV)
        right = lax.rem(my + 1, N_DEV)

        barrier_sem = pltpu.get_barrier_semaphore()
        for nbr in (left, right):
            pl.semaphore_signal(
                barrier_sem, inc=1,
                device_id=(nbr,), device_id_type=pl.DeviceIdType.MESH,
            )
        pl.semaphore_wait(barrier_sem, 2)

        def chunk(c):
            return x_ref[pl.ds(c * mc, mc), :].astype(jnp.bfloat16)

        rs_send[0, :, :] = chunk(my)
        for h in range(N_HOP):
            rdma = pltpu.make_async_remote_copy(
                src_ref=rs_send.at[h],
                dst_ref=rs_recv.at[h],
                send_sem=send_sems.at[h],
                recv_sem=recv_sems.at[h],
                device_id=(right,),
                device_id_type=pl.DeviceIdType.MESH,
            )
            rdma.start()
            rdma.wait()
            c_recv = lax.rem(my - h - 1 + N_DEV, N_DEV)
            acc = rs_recv[h, :, :] + chunk(c_recv)
            if h < N_HOP - 1:
                rs_send[h + 1, :, :] = acc
            else:
                ag_buf[0, :, :] = acc

        out_ref[pl.ds(right * mc, mc), :] = ag_buf[0, :, :].astype(jnp.float32)

        for t in range(N_HOP):
            rdma = pltpu.make_async_remote_copy(
                src_ref=ag_buf.at[t],
                dst_ref=ag_buf.at[t + 1],
                send_sem=send_sems.at[N_HOP + t],
                recv_sem=recv_sems.at[N_HOP + t],
                device_id=(right,),
                device_id_type=pl.DeviceIdType.MESH,
            )
            rdma.start()
            rdma.wait()
            c = lax.rem(my - t + N_DEV, N_DEV)
            out_ref[pl.ds(c * mc, mc), :] = ag_buf[t + 1, :, :].astype(jnp.float32)

    return pl.pallas_call(
        body,
        out_shape=jax.ShapeDtypeStruct((m, n), jnp.float32),
        in_specs=[pl.BlockSpec(memory_space=pltpu.VMEM)],
        out_specs=pl.BlockSpec(memory_space=pltpu.VMEM),
        scratch_shapes=[
            pltpu.VMEM((N_HOP, mc, n), jnp.bfloat16),
            pltpu.VMEM((N_HOP, mc, n), jnp.bfloat16),
            pltpu.VMEM((N_HOP + 1, mc, n), jnp.bfloat16),
            pltpu.SemaphoreType.DMA((2 * N_HOP,)),
            pltpu.SemaphoreType.DMA((2 * N_HOP,)),
        ],
        compiler_params=pltpu.CompilerParams(collective_id=0),
    )(x)


# device time: 15083 ns/iter; 1.5622x vs baseline; 1.5622x over previous
import jax
import jax.numpy as jnp
from jax import lax
from jax.experimental import pallas as pl
from jax.experimental.pallas import tpu as pltpu

N_DEV = 4


def kernel(x):
    m, n = x.shape
    mc = m // N_DEV

    def body(x_ref, out_ref, rs_send, rs_recv, ag_send, ag_recv,
             rs_send_sems, rs_recv_sems, ag_send_sems, ag_recv_sems):
        my = lax.axis_index("i")

        barrier_sem = pltpu.get_barrier_semaphore()
        for k in range(1, N_DEV):
            pl.semaphore_signal(
                barrier_sem, inc=1,
                device_id=(lax.rem(my + k, N_DEV),),
                device_id_type=pl.DeviceIdType.MESH,
            )
        pl.semaphore_wait(barrier_sem, N_DEV - 1)

        def chunk(c):
            return x_ref[pl.ds(c * mc, mc), :].astype(jnp.bfloat16)

        def copy(src_slot, dst_ref_slots, send_sem, recv_sem, k):
            return pltpu.make_async_remote_copy(
                src_ref=src_slot,
                dst_ref=dst_ref_slots,
                send_sem=send_sem,
                recv_sem=recv_sem,
                device_id=(lax.rem(my + k, N_DEV),),
                device_id_type=pl.DeviceIdType.MESH,
            )

        sends = []
        for k in (2, 1, 3):
            rs_send[k, :, :] = chunk(lax.rem(my + k, N_DEV))
            rdma = copy(rs_send.at[k], rs_recv.at[k],
                        rs_send_sems.at[k], rs_recv_sems.at[k], k)
            rdma.start()
            sends.append(rdma)

        rs_recv[0, :, :] = chunk(my)
        for r in (1, 3, 2):
            copy(rs_send.at[r], rs_recv.at[r],
                 rs_send_sems.at[r], rs_recv_sems.at[r], r).wait_recv()

        reduced = rs_recv[:, :, :].astype(jnp.float32).sum(axis=0)
        ag_send[:, :] = reduced.astype(jnp.bfloat16)
        out_ref[pl.ds(my * mc, mc), :] = reduced

        for k in (2, 1, 3):
            rdma = copy(ag_send, ag_recv.at[k],
                        ag_send_sems.at[k], ag_recv_sems.at[k], k)
            rdma.start()
            sends.append(rdma)

        for r in (1, 3, 2):
            copy(ag_send, ag_recv.at[r],
                 ag_send_sems.at[r], ag_recv_sems.at[r], r).wait_recv()
            c = lax.rem(my - r + N_DEV, N_DEV)
            out_ref[pl.ds(c * mc, mc), :] = ag_recv[r, :, :].astype(jnp.float32)

        for rdma in sends:
            rdma.wait_send()

    return pl.pallas_call(
        body,
        out_shape=jax.ShapeDtypeStruct((m, n), jnp.float32),
        in_specs=[pl.BlockSpec(memory_space=pltpu.VMEM)],
        out_specs=pl.BlockSpec(memory_space=pltpu.VMEM),
        scratch_shapes=[
            pltpu.VMEM((N_DEV, mc, n), jnp.bfloat16),
            pltpu.VMEM((N_DEV, mc, n), jnp.bfloat16),
            pltpu.VMEM((mc, n), jnp.bfloat16),
            pltpu.VMEM((N_DEV, mc, n), jnp.bfloat16),
            pltpu.SemaphoreType.DMA((N_DEV,)),
            pltpu.SemaphoreType.DMA((N_DEV,)),
            pltpu.SemaphoreType.DMA((N_DEV,)),
            pltpu.SemaphoreType.DMA((N_DEV,)),
        ],
        compiler_params=pltpu.CompilerParams(collective_id=0),
    )(x)


# device time: 13781 ns/iter; 1.7097x vs baseline; 1.0945x over previous
import jax
import jax.numpy as jnp
from jax import lax
from jax.experimental import pallas as pl
from jax.experimental.pallas import tpu as pltpu

N_DEV = 4
SUB = 2


def kernel(x):
    m, n = x.shape
    mc = m // N_DEV
    ms = mc // SUB

    def body(x_ref, out_ref, xb, rs_recv, ag_send, ag_recv,
             rs_send_sems, rs_recv_sems, ag_send_sems, ag_recv_sems):
        my = lax.axis_index("i")

        barrier_sem = pltpu.get_barrier_semaphore()
        for k in range(1, N_DEV):
            pl.semaphore_signal(
                barrier_sem, inc=1,
                device_id=(lax.rem(my + k, N_DEV),),
                device_id_type=pl.DeviceIdType.MESH,
            )
        pl.semaphore_wait(barrier_sem, N_DEV - 1)

        xb[:, :] = x_ref[:, :].astype(jnp.bfloat16)

        def rs_copy(s, k):
            start = lax.rem(my + k, N_DEV) * mc + s * ms
            return pltpu.make_async_remote_copy(
                src_ref=xb.at[pl.ds(start, ms), :],
                dst_ref=rs_recv.at[s, k],
                send_sem=rs_send_sems.at[s, k],
                recv_sem=rs_recv_sems.at[s, k],
                device_id=(lax.rem(my + k, N_DEV),),
                device_id_type=pl.DeviceIdType.MESH,
            )

        def ag_copy(s, k):
            return pltpu.make_async_remote_copy(
                src_ref=ag_send.at[s],
                dst_ref=ag_recv.at[s, k],
                send_sem=ag_send_sems.at[s, k],
                recv_sem=ag_recv_sems.at[s, k],
                device_id=(lax.rem(my + k, N_DEV),),
                device_id_type=pl.DeviceIdType.MESH,
            )

        sends = []
        for s in range(SUB):
            for k in (2, 1, 3):
                rdma = rs_copy(s, k)
                rdma.start()
                sends.append(rdma)

        for s in range(SUB):
            rs_recv[s, 0, :, :] = xb[pl.ds(my * mc + s * ms, ms), :]

        for s in range(SUB):
            for r in (1, 3, 2):
                rs_copy(s, r).wait_recv()
            red = rs_recv[s, :, :, :].astype(jnp.float32).sum(axis=0)
            ag_send[s, :, :] = red.astype(jnp.bfloat16)
            out_ref[pl.ds(my * mc + s * ms, ms), :] = red
            for k in (2, 1, 3):
                rdma = ag_copy(s, k)
                rdma.start()
                sends.append(rdma)

        for s in range(SUB):
            for r in (1, 3, 2):
                ag_copy(s, r).wait_recv()
                c = lax.rem(my - r + N_DEV, N_DEV)
                out_ref[pl.ds(c * mc + s * ms, ms), :] = (
                    ag_recv[s, r, :, :].astype(jnp.float32))

        for rdma in sends:
            rdma.wait_send()

    return pl.pallas_call(
        body,
        out_shape=jax.ShapeDtypeStruct((m, n), jnp.float32),
        in_specs=[pl.BlockSpec(memory_space=pltpu.VMEM)],
        out_specs=pl.BlockSpec(memory_space=pltpu.VMEM),
        scratch_shapes=[
            pltpu.VMEM((m, n), jnp.bfloat16),
            pltpu.VMEM((SUB, N_DEV, ms, n), jnp.bfloat16),
            pltpu.VMEM((SUB, ms, n), jnp.bfloat16),
            pltpu.VMEM((SUB, N_DEV, ms, n), jnp.bfloat16),
            pltpu.SemaphoreType.DMA((SUB, N_DEV)),
            pltpu.SemaphoreType.DMA((SUB, N_DEV)),
            pltpu.SemaphoreType.DMA((SUB, N_DEV)),
            pltpu.SemaphoreType.DMA((SUB, N_DEV)),
        ],
        compiler_params=pltpu.CompilerParams(collective_id=0),
    )(x)


# device time: 12694 ns/iter; 1.8562x vs baseline; 1.0856x over previous
import jax
import jax.numpy as jnp
from jax import lax
from jax.experimental import pallas as pl
from jax.experimental.pallas import tpu as pltpu

N_DEV = 4
SUB = 4


def kernel(x):
    m, n = x.shape
    mc = m // N_DEV
    ms = mc // SUB

    def body(x_ref, out_ref, xb, rs_recv, ag_send, ag_recv,
             rs_send_sems, rs_recv_sems, ag_send_sems, ag_recv_sems):
        my = lax.axis_index("i")

        barrier_sem = pltpu.get_barrier_semaphore()
        for k in range(1, N_DEV):
            pl.semaphore_signal(
                barrier_sem, inc=1,
                device_id=(lax.rem(my + k, N_DEV),),
                device_id_type=pl.DeviceIdType.MESH,
            )
        pl.semaphore_wait(barrier_sem, N_DEV - 1)

        xb[:, :] = x_ref[:, :].astype(jnp.bfloat16)

        def rs_copy(s, k):
            start = lax.rem(my + k, N_DEV) * mc + s * ms
            return pltpu.make_async_remote_copy(
                src_ref=xb.at[pl.ds(start, ms), :],
                dst_ref=rs_recv.at[s, k],
                send_sem=rs_send_sems.at[s, k],
                recv_sem=rs_recv_sems.at[s, k],
                device_id=(lax.rem(my + k, N_DEV),),
                device_id_type=pl.DeviceIdType.MESH,
            )

        def ag_copy(s, k):
            return pltpu.make_async_remote_copy(
                src_ref=ag_send.at[s],
                dst_ref=ag_recv.at[s, k],
                send_sem=ag_send_sems.at[s, k],
                recv_sem=ag_recv_sems.at[s, k],
                device_id=(lax.rem(my + k, N_DEV),),
                device_id_type=pl.DeviceIdType.MESH,
            )

        sends = []
        for s in range(SUB):
            for k in (2, 1, 3):
                rdma = rs_copy(s, k)
                rdma.start()
                sends.append(rdma)

        for s in range(SUB):
            rs_recv[s, 0, :, :] = xb[pl.ds(my * mc + s * ms, ms), :]

        for s in range(SUB):
            for r in (1, 3, 2):
                rs_copy(s, r).wait_recv()
            red = rs_recv[s, :, :, :].astype(jnp.float32).sum(axis=0)
            ag_send[s, :, :] = red.astype(jnp.bfloat16)
            out_ref[pl.ds(my * mc + s * ms, ms), :] = red
            for k in (2, 1, 3):
                rdma = ag_copy(s, k)
                rdma.start()
                sends.append(rdma)

        for s in range(SUB):
            for r in (1, 3, 2):
                ag_copy(s, r).wait_recv()
                c = lax.rem(my - r + N_DEV, N_DEV)
                out_ref[pl.ds(c * mc + s * ms, ms), :] = (
                    ag_recv[s, r, :, :].astype(jnp.float32))

        for rdma in sends:
            rdma.wait_send()

    return pl.pallas_call(
        body,
        out_shape=jax.ShapeDtypeStruct((m, n), jnp.float32),
        in_specs=[pl.BlockSpec(memory_space=pltpu.VMEM)],
        out_specs=pl.BlockSpec(memory_space=pltpu.VMEM),
        scratch_shapes=[
            pltpu.VMEM((m, n), jnp.bfloat16),
            pltpu.VMEM((SUB, N_DEV, ms, n), jnp.bfloat16),
            pltpu.VMEM((SUB, ms, n), jnp.bfloat16),
            pltpu.VMEM((SUB, N_DEV, ms, n), jnp.bfloat16),
            pltpu.SemaphoreType.DMA((SUB, N_DEV)),
            pltpu.SemaphoreType.DMA((SUB, N_DEV)),
            pltpu.SemaphoreType.DMA((SUB, N_DEV)),
            pltpu.SemaphoreType.DMA((SUB, N_DEV)),
        ],
        compiler_params=pltpu.CompilerParams(collective_id=0),
    )(x)
